# baseline (speedup 1.0000x reference)
import jax
import jax.numpy as jnp
from jax.experimental import pallas as pl

_B, _SEG, _D, _H, _O = 16, 2048, 1024, 256, 32


def _readout_body(x_ref, w1_ref, w2_ref, o_ref):
    x = x_ref[...]
    xb = x.astype(jnp.bfloat16)
    t = jnp.tanh(jnp.dot(xb, w1_ref[...].astype(jnp.bfloat16).T,
                         preferred_element_type=jnp.float32))
    s = jnp.dot(t, w2_ref[...].T)
    s = s - jnp.max(s, axis=0, keepdims=True)
    e = jnp.exp(s)
    attn = e / jnp.sum(e, axis=0, keepdims=True)
    o_ref[...] = jax.lax.dot_general(attn, x, (((0,), (0,)), ((), ())))


def kernel(embeddings, scope, w1, w2):
    del scope
    out = pl.pallas_call(
        _readout_body,
        grid=(_B,),
        in_specs=[
            pl.BlockSpec((_SEG, _D), lambda b: (b, 0)),
            pl.BlockSpec((_H, _D), lambda b: (0, 0)),
            pl.BlockSpec((_O, _H), lambda b: (0, 0)),
        ],
        out_specs=pl.BlockSpec((_O, _D), lambda b: (b, 0)),
        out_shape=jax.ShapeDtypeStruct((_B * _O, _D), jnp.float32),
    )(embeddings, w1, w2)
    return out.reshape(_B, _O * _D)


# speedup vs baseline: 1.0025x; 1.0025x over previous
import jax
import jax.numpy as jnp
from jax.experimental import pallas as pl

_B, _SEG, _D, _H, _O = 16, 2048, 1024, 256, 32


def _readout_body(x_ref, w1_ref, w2_ref, o_ref):
    x = x_ref[...]
    xb = x.astype(jnp.bfloat16)
    t = jnp.tanh(jnp.dot(xb, w1_ref[...].astype(jnp.bfloat16).T,
                         preferred_element_type=jnp.float32))
    s = jnp.dot(t.astype(jnp.bfloat16), w2_ref[...].astype(jnp.bfloat16).T,
                preferred_element_type=jnp.float32)
    s = s - jnp.max(s, axis=0, keepdims=True)
    e = jnp.exp(s)
    attn = (e / jnp.sum(e, axis=0, keepdims=True)).astype(jnp.bfloat16)
    o_ref[...] = jax.lax.dot_general(
        attn, xb, (((0,), (0,)), ((), ())),
        preferred_element_type=jnp.float32)


def kernel(embeddings, scope, w1, w2):
    del scope
    out = pl.pallas_call(
        _readout_body,
        grid=(_B,),
        in_specs=[
            pl.BlockSpec((_SEG, _D), lambda b: (b, 0)),
            pl.BlockSpec((_H, _D), lambda b: (0, 0)),
            pl.BlockSpec((_O, _H), lambda b: (0, 0)),
        ],
        out_specs=pl.BlockSpec((_O, _D), lambda b: (b, 0)),
        out_shape=jax.ShapeDtypeStruct((_B * _O, _D), jnp.float32),
    )(embeddings, w1, w2)
    return out.reshape(_B, _O * _D)


# speedup vs baseline: 1.1055x; 1.1028x over previous
import jax
import jax.numpy as jnp
from jax.experimental import pallas as pl

_B, _SEG, _D, _H, _O = 16, 2048, 1024, 256, 32


def _readout_body(x_ref, w1_ref, w2_ref, o_ref):
    x = x_ref[...]
    xb = x.astype(jnp.bfloat16)
    w2 = w2_ref[...]
    t = jnp.tanh(jnp.dot(xb, w1_ref[...].astype(jnp.bfloat16).T,
                         preferred_element_type=jnp.float32))
    s = jnp.dot(t.astype(jnp.bfloat16), w2.astype(jnp.bfloat16).T,
                preferred_element_type=jnp.float32)
    k = jnp.sum(jnp.abs(w2), axis=1)
    e = jnp.exp(s - k[None, :])
    l = jnp.sum(e, axis=0)
    acc = jax.lax.dot_general(
        e.astype(jnp.bfloat16), xb, (((0,), (0,)), ((), ())),
        preferred_element_type=jnp.float32)
    o_ref[...] = acc / l[:, None]


def kernel(embeddings, scope, w1, w2):
    del scope
    out = pl.pallas_call(
        _readout_body,
        grid=(_B,),
        in_specs=[
            pl.BlockSpec((_SEG, _D), lambda b: (b, 0)),
            pl.BlockSpec((_H, _D), lambda b: (0, 0)),
            pl.BlockSpec((_O, _H), lambda b: (0, 0)),
        ],
        out_specs=pl.BlockSpec((_O, _D), lambda b: (b, 0)),
        out_shape=jax.ShapeDtypeStruct((_B * _O, _D), jnp.float32),
    )(embeddings, w1, w2)
    return out.reshape(_B, _O * _D)
